# Initial kernel scaffold; baseline (speedup 1.0000x reference)
import jax
import jax.numpy as jnp
from jax.experimental import pallas as pl


def kernel(x, edge_index, W1, b1, W2, b2):
    raise NotImplementedError("write your pallas kernel here")


# speedup vs baseline: 75.0128x; 75.0128x over previous
import functools

import jax
import jax.numpy as jnp
from jax import lax
from jax.experimental import pallas as pl
from jax.experimental.pallas import tpu as pltpu
from jax.experimental.pallas import tpu_sc as plsc

f32 = jnp.float32
i32 = jnp.int32

N = 50000
NPAD = 50176
R = NPAD // 128
E = 800000
NC, NS = 2, 16
NT = NC * NS
EPT = 25600
EPAD = EPT * NT
ERows = EPAD // 128
RPT = EPT // 128
SL = NPAD // NS

_mesh = plsc.VectorSubcoreMesh(
    core_axis_name="c", subcore_axis_name="s", num_cores=NC, num_subcores=NS)


def _zero_acc_slice(zbuf, acc, sid):
    def zb(i, _):
        zbuf[pl.ds(i * 16, 16)] = jnp.zeros((16,), f32)
        return 0
    lax.fori_loop(0, SL // 16, zb, 0)
    pltpu.sync_copy(zbuf.at[pl.ds(0, SL)], acc.at[pl.ds(sid * SL, SL)])


@functools.partial(
    pl.kernel,
    out_type=jax.ShapeDtypeStruct((NC * NPAD,), f32),
    mesh=_mesh,
    scratch_types=[
        pltpu.VMEM((RPT, 128), i32),
        pltpu.VMEM((1, 128), f32),
        pltpu.VMEM((SL,), f32),
        pltpu.VMEM_SHARED((NPAD,), f32),
    ],
)
def _deg_kernel(dst2d, ones_row, out, dbuf, obuf, zbuf, acc):
    cid = lax.axis_index("c")
    sid = lax.axis_index("s")
    wid = cid * NS + sid
    _zero_acc_slice(zbuf, acc, sid)
    pltpu.sync_copy(ones_row, obuf)
    pltpu.sync_copy(dst2d.at[pl.ds(wid * RPT, RPT)], dbuf)
    plsc.subcore_barrier()

    def chunk(j, _):
        pltpu.sync_copy(obuf.at[0], acc.at[dbuf.at[j]], add=True)
        return 0
    lax.fori_loop(0, RPT, chunk, 0)
    plsc.subcore_barrier()
    pltpu.sync_copy(acc.at[pl.ds(sid * SL, SL)], zbuf.at[pl.ds(0, SL)])
    pltpu.sync_copy(zbuf.at[pl.ds(0, SL)],
                    out.at[pl.ds(cid * NPAD + sid * SL, SL)])


@functools.partial(
    pl.kernel,
    out_type=jax.ShapeDtypeStruct((NC * NPAD,), f32),
    mesh=_mesh,
    compiler_params=pltpu.CompilerParams(needs_layout_passes=False),
    scratch_types=[
        pltpu.VMEM((NPAD,), f32),
        pltpu.VMEM((EPT,), i32),
        pltpu.VMEM((RPT, 128), i32),
        pltpu.VMEM((EPT,), f32),
        pltpu.VMEM_SHARED((NPAD,), f32),
    ],
)
def _edge_kernel(table, src1d, dst2d, out, tab, sbuf, dbuf, vbuf, acc):
    cid = lax.axis_index("c")
    sid = lax.axis_index("s")
    wid = cid * NS + sid
    _zero_acc_slice(vbuf, acc, sid)
    pltpu.sync_copy(table, tab)
    pltpu.sync_copy(src1d.at[pl.ds(wid * EPT, EPT)], sbuf)
    pltpu.sync_copy(dst2d.at[pl.ds(wid * RPT, RPT)], dbuf)
    plsc.subcore_barrier()

    def gb(i, _):
        idx = sbuf[pl.ds(i * 16, 16)]
        vbuf[pl.ds(i * 16, 16)] = plsc.load_gather(tab, [idx])
        return 0
    lax.fori_loop(0, EPT // 16, gb, 0)

    def chunk(j, _):
        pltpu.sync_copy(vbuf.at[pl.ds(j * 128, 128)], acc.at[dbuf.at[j]],
                        add=True)
        return 0
    lax.fori_loop(0, RPT, chunk, 0)
    plsc.subcore_barrier()
    pltpu.sync_copy(acc.at[pl.ds(sid * SL, SL)], vbuf.at[pl.ds(0, SL)])
    pltpu.sync_copy(vbuf.at[pl.ds(0, SL)],
                    out.at[pl.ds(cid * NPAD + sid * SL, SL)])


def _t1_body(d0, d1, xr, a_o, dis_o, s1self_o, dis2_o):
    deg = d0[...] + d1[...] + 1.0
    dis = lax.rsqrt(deg)
    dis = dis * (1.5 - 0.5 * deg * dis * dis)
    dis2 = 1.0 / deg
    dis_o[...] = dis
    dis2_o[...] = dis2
    a_o[...] = xr[...] * dis
    s1self_o[...] = xr[...] * dis2


_t1 = pl.pallas_call(
    _t1_body,
    out_shape=[jax.ShapeDtypeStruct((R, 128), f32)] * 4,
)

B2 = 512


def _t2_body(p0, p1, dis, s1self, dis2, w1, b1r, w2, b_o, zself_o):
    s1 = dis[...] * (p0[...] + p1[...]) + s1self[...]
    h = jnp.maximum(s1 * w1[...] + b1r[...], 0.0)
    z = jnp.sum(h * w2[...], axis=1, keepdims=True)
    b_o[...] = z * dis[...]
    zself_o[...] = z * dis2[...]


_col = pl.BlockSpec((B2, 1), lambda i: (i, 0))
_row = pl.BlockSpec((1, 128), lambda i: (0, 0))
_t2 = pl.pallas_call(
    _t2_body,
    grid=(NPAD // B2,),
    in_specs=[_col, _col, _col, _col, _col, _row, _row, _row],
    out_specs=[_col, _col],
    out_shape=[jax.ShapeDtypeStruct((NPAD, 1), f32)] * 2,
)


def _t3_body(o0, o1, dis, zself, b2s, out_o):
    out_o[...] = dis[...] * (o0[...] + o1[...]) + zself[...] + b2s[0, 0]


_t3 = pl.pallas_call(
    _t3_body,
    in_specs=[pl.BlockSpec(memory_space=pltpu.VMEM)] * 4
    + [pl.BlockSpec(memory_space=pltpu.SMEM)],
    out_shape=jax.ShapeDtypeStruct((R, 128), f32),
)


def kernel(x, edge_index, W1, b1, W2, b2):
    x = x.astype(f32)
    ei = edge_index.astype(i32)
    pad_e = jnp.full((EPAD - E,), NPAD - 1, i32)
    src1d = jnp.concatenate([ei[0], pad_e])
    dst2d = jnp.concatenate([ei[1], pad_e]).reshape(ERows, 128)
    xp = jnp.pad(x[:, 0], (0, NPAD - N)).reshape(R, 128)
    ones_row = jnp.ones((1, 128), f32)

    degp = _deg_kernel(dst2d, ones_row).reshape(NC, NPAD)
    a, dis, s1self, dis2 = _t1(degp[0].reshape(R, 128),
                               degp[1].reshape(R, 128), xp)

    s1p = _edge_kernel(a.reshape(NPAD), src1d, dst2d).reshape(NC, NPAD)
    bvec, zself = _t2(s1p[0].reshape(NPAD, 1), s1p[1].reshape(NPAD, 1),
                      dis.reshape(NPAD, 1), s1self.reshape(NPAD, 1),
                      dis2.reshape(NPAD, 1),
                      W1, b1.reshape(1, 128), W2.reshape(1, 128))

    outp = _edge_kernel(bvec.reshape(NPAD), src1d, dst2d).reshape(NC, NPAD)
    out = _t3(outp[0].reshape(R, 128), outp[1].reshape(R, 128), dis,
              zself.reshape(R, 128), b2.reshape(1, 1))
    return out.reshape(NPAD)[:N][:, None]


# speedup vs baseline: 124.4008x; 1.6584x over previous
import functools

import jax
import jax.numpy as jnp
from jax import lax
from jax.experimental import pallas as pl
from jax.experimental.pallas import tpu as pltpu
from jax.experimental.pallas import tpu_sc as plsc

f32 = jnp.float32
i32 = jnp.int32

N = 50000
NPAD = 50176
R = NPAD // 128
E = 800000
NC, NS = 2, 16
NT = NC * NS
EPT = 25600
EPAD = EPT * NT
ERows = EPAD // 128
RPT = EPT // 128
SL = NPAD // NS

_mesh = plsc.VectorSubcoreMesh(
    core_axis_name="c", subcore_axis_name="s", num_cores=NC, num_subcores=NS)


def _zero_acc_slice(zbuf, acc, sid):
    def zb(i, _):
        zbuf[pl.ds(i * 16, 16)] = jnp.zeros((16,), f32)
        return 0
    lax.fori_loop(0, SL // 16, zb, 0)
    pltpu.sync_copy(zbuf.at[pl.ds(0, SL)], acc.at[pl.ds(sid * SL, SL)])


@functools.partial(
    pl.kernel,
    out_type=jax.ShapeDtypeStruct((NC * NPAD,), f32),
    mesh=_mesh,
    scratch_types=[
        pltpu.VMEM((RPT, 128), i32),
        pltpu.VMEM((1, 128), f32),
        pltpu.VMEM((SL,), f32),
        pltpu.VMEM_SHARED((NPAD,), f32),
    ],
)
def _deg_kernel(dst2d, ones_row, out, dbuf, obuf, zbuf, acc):
    cid = lax.axis_index("c")
    sid = lax.axis_index("s")
    wid = cid * NS + sid
    _zero_acc_slice(zbuf, acc, sid)
    pltpu.sync_copy(ones_row, obuf)
    pltpu.sync_copy(dst2d.at[pl.ds(wid * RPT, RPT)], dbuf)
    plsc.subcore_barrier()

    def chunk(j, _):
        pltpu.sync_copy(obuf.at[0], acc.at[dbuf.at[j]], add=True)
        return 0
    lax.fori_loop(0, RPT, chunk, 0)
    plsc.subcore_barrier()
    pltpu.sync_copy(acc.at[pl.ds(sid * SL, SL)], zbuf.at[pl.ds(0, SL)])
    pltpu.sync_copy(zbuf.at[pl.ds(0, SL)],
                    out.at[pl.ds(cid * NPAD + sid * SL, SL)])


@functools.partial(
    pl.kernel,
    out_type=jax.ShapeDtypeStruct((NC * NPAD,), f32),
    mesh=_mesh,
    compiler_params=pltpu.CompilerParams(needs_layout_passes=False),
    scratch_types=[
        pltpu.VMEM((NPAD,), f32),
        pltpu.VMEM((EPT,), i32),
        pltpu.VMEM((RPT, 128), i32),
        pltpu.VMEM((EPT,), f32),
        pltpu.VMEM_SHARED((NPAD,), f32),
    ],
)
def _edge_kernel(table, src1d, dst2d, out, tab, sbuf, dbuf, vbuf, acc):
    cid = lax.axis_index("c")
    sid = lax.axis_index("s")
    wid = cid * NS + sid
    _zero_acc_slice(vbuf, acc, sid)
    pltpu.sync_copy(table, tab)
    pltpu.sync_copy(src1d.at[pl.ds(wid * EPT, EPT)], sbuf)
    pltpu.sync_copy(dst2d.at[pl.ds(wid * RPT, RPT)], dbuf)
    plsc.subcore_barrier()

    def gb(i, _):
        idx = sbuf[pl.ds(i * 16, 16)]
        vbuf[pl.ds(i * 16, 16)] = plsc.load_gather(tab, [idx])
        return 0
    lax.fori_loop(0, EPT // 16, gb, 0)

    def chunk(j, _):
        pltpu.sync_copy(vbuf.at[pl.ds(j * 128, 128)], acc.at[dbuf.at[j]],
                        add=True)
        return 0
    lax.fori_loop(0, RPT, chunk, 0)
    plsc.subcore_barrier()
    pltpu.sync_copy(acc.at[pl.ds(sid * SL, SL)], vbuf.at[pl.ds(0, SL)])
    pltpu.sync_copy(vbuf.at[pl.ds(0, SL)],
                    out.at[pl.ds(cid * NPAD + sid * SL, SL)])


def _t1_body(d0, d1, xr, a_o, dis_o, s1self_o, dis2_o):
    deg = d0[...] + d1[...] + 1.0
    dis = lax.rsqrt(deg)
    dis = dis * (1.5 - 0.5 * deg * dis * dis)
    dis2 = 1.0 / deg
    dis_o[...] = dis
    dis2_o[...] = dis2
    a_o[...] = xr[...] * dis
    s1self_o[...] = xr[...] * dis2


_t1 = pl.pallas_call(
    _t1_body,
    out_shape=[jax.ShapeDtypeStruct((R, 128), f32)] * 4,
)

RB = 56


def _t2_body(p0, p1, dis, s1self, dis2, w1s, b1s, w2s, b_o, zself_o):
    s1 = dis[...] * (p0[...] + p1[...]) + s1self[...]

    def step(k, z):
        h = jnp.maximum(s1 * w1s[0, k] + b1s[0, k], 0.0)
        return z + h * w2s[0, k]
    z = lax.fori_loop(0, 128, step, jnp.zeros_like(s1))
    b_o[...] = z * dis[...]
    zself_o[...] = z * dis2[...]


_blk = pl.BlockSpec((RB, 128), lambda i: (i, 0))
_wrow = pl.BlockSpec(memory_space=pltpu.SMEM)
_t2 = pl.pallas_call(
    _t2_body,
    grid=(R // RB,),
    in_specs=[_blk, _blk, _blk, _blk, _blk, _wrow, _wrow, _wrow],
    out_specs=[_blk, _blk],
    out_shape=[jax.ShapeDtypeStruct((R, 128), f32)] * 2,
)


def _t3_body(o0, o1, dis, zself, b2s, out_o):
    out_o[...] = dis[...] * (o0[...] + o1[...]) + zself[...] + b2s[0, 0]


_t3 = pl.pallas_call(
    _t3_body,
    in_specs=[pl.BlockSpec(memory_space=pltpu.VMEM)] * 4
    + [pl.BlockSpec(memory_space=pltpu.SMEM)],
    out_shape=jax.ShapeDtypeStruct((R, 128), f32),
)


def kernel(x, edge_index, W1, b1, W2, b2):
    x = x.astype(f32)
    ei = edge_index.astype(i32)
    pad_e = jnp.full((EPAD - E,), NPAD - 1, i32)
    src1d = jnp.concatenate([ei[0], pad_e])
    dst2d = jnp.concatenate([ei[1], pad_e]).reshape(ERows, 128)
    xp = jnp.pad(x[:, 0], (0, NPAD - N)).reshape(R, 128)
    ones_row = jnp.ones((1, 128), f32)

    degp = _deg_kernel(dst2d, ones_row).reshape(NC, NPAD)
    a, dis, s1self, dis2 = _t1(degp[0].reshape(R, 128),
                               degp[1].reshape(R, 128), xp)

    s1p = _edge_kernel(a.reshape(NPAD), src1d, dst2d).reshape(NC, NPAD)
    bvec, zself = _t2(s1p[0].reshape(R, 128), s1p[1].reshape(R, 128),
                      dis, s1self, dis2,
                      W1, b1.reshape(1, 128), W2.reshape(1, 128))

    outp = _edge_kernel(bvec.reshape(NPAD), src1d, dst2d).reshape(NC, NPAD)
    out = _t3(outp[0].reshape(R, 128), outp[1].reshape(R, 128), dis,
              zself, b2.reshape(1, 1))
    return out.reshape(NPAD)[:N][:, None]


# speedup vs baseline: 210.7784x; 1.6943x over previous
import functools

import jax
import jax.numpy as jnp
from jax import lax
from jax.experimental import pallas as pl
from jax.experimental.pallas import tpu as pltpu
from jax.experimental.pallas import tpu_sc as plsc

f32 = jnp.float32
i32 = jnp.int32

N = 50000
NPAD = 50176
R = NPAD // 128
E = 800000
NC, NS = 2, 16
NT = NC * NS
EPT = 25088
RPT = EPT // 128
LAST_START = E - EPT
SKIP_LAST = 22
SL = NPAD // NS

_mesh = plsc.VectorSubcoreMesh(
    core_axis_name="c", subcore_axis_name="s", num_cores=NC, num_subcores=NS)


def _tile_window(cid, sid):
    wid = cid * NS + sid
    start = jnp.minimum(wid * EPT, LAST_START)
    row_lo = jnp.where(wid == NT - 1, SKIP_LAST, 0)
    return wid, start, row_lo


def _zero_acc_slice(zbuf, acc, sid):
    def zb(i, _):
        zbuf[pl.ds(i * 16, 16)] = jnp.zeros((16,), f32)
        return 0
    lax.fori_loop(0, SL // 16, zb, 0)
    pltpu.sync_copy(zbuf.at[pl.ds(0, SL)], acc.at[pl.ds(sid * SL, SL)])


@functools.partial(
    pl.kernel,
    out_type=jax.ShapeDtypeStruct((NC * NPAD,), f32),
    mesh=_mesh,
    compiler_params=pltpu.CompilerParams(needs_layout_passes=False),
    scratch_types=[
        pltpu.VMEM((2, EPT), i32),
        pltpu.VMEM((1, 128), f32),
        pltpu.VMEM((SL,), f32),
        pltpu.VMEM_SHARED((NPAD,), f32),
        pltpu.SemaphoreType.DMA,
        pltpu.SemaphoreType.DMA,
    ],
)
def _deg_kernel(ei, ones_row, out, ebuf, obuf, zbuf, acc, esem, ssem):
    cid = lax.axis_index("c")
    sid = lax.axis_index("s")
    wid, start, row_lo = _tile_window(cid, sid)
    edma = pltpu.async_copy(ei.at[:, pl.ds(start, EPT)], ebuf, esem)
    pltpu.sync_copy(ones_row, obuf)
    _zero_acc_slice(zbuf, acc, sid)
    edma.wait()
    plsc.subcore_barrier()

    def fire(j, _):
        pltpu.async_copy(obuf.at[0], acc.at[ebuf.at[1, pl.ds(j * 128, 128)]],
                         ssem, add=True)
        return 0
    lax.fori_loop(row_lo, RPT, fire, 0)

    def drain(j, _):
        pltpu.make_async_copy(obuf.at[0],
                              acc.at[ebuf.at[1, pl.ds(j * 128, 128)]],
                              ssem).wait()
        return 0
    lax.fori_loop(row_lo, RPT, drain, 0)
    plsc.subcore_barrier()
    pltpu.sync_copy(acc.at[pl.ds(sid * SL, SL)], zbuf.at[pl.ds(0, SL)])
    pltpu.sync_copy(zbuf.at[pl.ds(0, SL)],
                    out.at[pl.ds(cid * NPAD + sid * SL, SL)])


@functools.partial(
    pl.kernel,
    out_type=jax.ShapeDtypeStruct((NC * NPAD,), f32),
    mesh=_mesh,
    compiler_params=pltpu.CompilerParams(needs_layout_passes=False),
    scratch_types=[
        pltpu.VMEM((NPAD,), f32),
        pltpu.VMEM((2, EPT), i32),
        pltpu.VMEM((EPT,), f32),
        pltpu.VMEM_SHARED((NPAD,), f32),
        pltpu.SemaphoreType.DMA,
        pltpu.SemaphoreType.DMA,
        pltpu.SemaphoreType.DMA,
    ],
)
def _edge_kernel(table, ei, out, tab, ebuf, vbuf, acc, esem, tsem, ssem):
    cid = lax.axis_index("c")
    sid = lax.axis_index("s")
    wid, start, row_lo = _tile_window(cid, sid)
    edma = pltpu.async_copy(ei.at[:, pl.ds(start, EPT)], ebuf, esem)
    tdma = pltpu.async_copy(table, tab, tsem)
    _zero_acc_slice(vbuf, acc, sid)
    edma.wait()
    tdma.wait()

    def gb(i, _):
        for u in range(4):
            idx = ebuf[0, pl.ds(i * 64 + u * 16, 16)]
            vbuf[pl.ds(i * 64 + u * 16, 16)] = plsc.load_gather(tab, [idx])
        return 0
    lax.fori_loop(row_lo * 2, RPT * 2, gb, 0)
    plsc.subcore_barrier()

    def fire(j, _):
        pltpu.async_copy(vbuf.at[pl.ds(j * 128, 128)],
                         acc.at[ebuf.at[1, pl.ds(j * 128, 128)]],
                         ssem, add=True)
        return 0
    lax.fori_loop(row_lo, RPT, fire, 0)

    def drain(j, _):
        pltpu.make_async_copy(vbuf.at[pl.ds(j * 128, 128)],
                              acc.at[ebuf.at[1, pl.ds(j * 128, 128)]],
                              ssem).wait()
        return 0
    lax.fori_loop(row_lo, RPT, drain, 0)
    plsc.subcore_barrier()
    pltpu.sync_copy(acc.at[pl.ds(sid * SL, SL)], vbuf.at[pl.ds(0, SL)])
    pltpu.sync_copy(vbuf.at[pl.ds(0, SL)],
                    out.at[pl.ds(cid * NPAD + sid * SL, SL)])


def _t1_body(d0, d1, xr, a_o, dis_o, s1self_o, dis2_o):
    deg = d0[...] + d1[...] + 1.0
    dis = lax.rsqrt(deg)
    dis = dis * (1.5 - 0.5 * deg * dis * dis)
    dis2 = 1.0 / deg
    dis_o[...] = dis
    dis2_o[...] = dis2
    a_o[...] = xr[...] * dis
    s1self_o[...] = xr[...] * dis2


_t1 = pl.pallas_call(
    _t1_body,
    out_shape=[jax.ShapeDtypeStruct((R, 128), f32)] * 4,
)

RB = 56


def _t2_body(p0, p1, dis, s1self, dis2, w1s, b1s, w2s, b_o, zself_o):
    s1 = dis[...] * (p0[...] + p1[...]) + s1self[...]

    def step(k, z):
        h = jnp.maximum(s1 * w1s[0, k] + b1s[0, k], 0.0)
        return z + h * w2s[0, k]
    z = lax.fori_loop(0, 128, step, jnp.zeros_like(s1))
    b_o[...] = z * dis[...]
    zself_o[...] = z * dis2[...]


_blk = pl.BlockSpec((RB, 128), lambda i: (i, 0))
_wrow = pl.BlockSpec(memory_space=pltpu.SMEM)
_t2 = pl.pallas_call(
    _t2_body,
    grid=(R // RB,),
    in_specs=[_blk, _blk, _blk, _blk, _blk, _wrow, _wrow, _wrow],
    out_specs=[_blk, _blk],
    out_shape=[jax.ShapeDtypeStruct((R, 128), f32)] * 2,
)


def _t3_body(o0, o1, dis, zself, b2s, out_o):
    out_o[...] = dis[...] * (o0[...] + o1[...]) + zself[...] + b2s[0, 0]


_t3 = pl.pallas_call(
    _t3_body,
    in_specs=[pl.BlockSpec(memory_space=pltpu.VMEM)] * 4
    + [pl.BlockSpec(memory_space=pltpu.SMEM)],
    out_shape=jax.ShapeDtypeStruct((R, 128), f32),
)


def kernel(x, edge_index, W1, b1, W2, b2):
    x = x.astype(f32)
    ei = edge_index.astype(i32)
    xp = jnp.pad(x[:, 0], (0, NPAD - N)).reshape(R, 128)
    ones_row = jnp.ones((1, 128), f32)

    degp = _deg_kernel(ei, ones_row).reshape(NC, NPAD)
    a, dis, s1self, dis2 = _t1(degp[0].reshape(R, 128),
                               degp[1].reshape(R, 128), xp)

    s1p = _edge_kernel(a.reshape(NPAD), ei).reshape(NC, NPAD)
    bvec, zself = _t2(s1p[0].reshape(R, 128), s1p[1].reshape(R, 128),
                      dis, s1self, dis2,
                      W1, b1.reshape(1, 128), W2.reshape(1, 128))

    outp = _edge_kernel(bvec.reshape(NPAD), ei).reshape(NC, NPAD)
    out = _t3(outp[0].reshape(R, 128), outp[1].reshape(R, 128), dis,
              zself, b2.reshape(1, 1))
    return out.reshape(NPAD)[:N][:, None]


# speedup vs baseline: 262.9737x; 1.2476x over previous
import functools

import jax
import jax.numpy as jnp
from jax import lax
from jax.experimental import pallas as pl
from jax.experimental.pallas import tpu as pltpu
from jax.experimental.pallas import tpu_sc as plsc

f32 = jnp.float32
i32 = jnp.int32

N = 50000
NPAD = 50176
R = NPAD // 128
E = 800000
NC, NS = 2, 16
NT = NC * NS
EPT = 25088
RPT = EPT // 128
LAST_START = E - EPT
SKIP_LAST = 22
SL = NPAD // NS

_mesh = plsc.VectorSubcoreMesh(
    core_axis_name="c", subcore_axis_name="s", num_cores=NC, num_subcores=NS)


def _tile_window(cid, sid):
    wid = cid * NS + sid
    start = jnp.minimum(wid * EPT, LAST_START)
    row_lo = jnp.where(wid == NT - 1, SKIP_LAST, 0)
    return wid, start, row_lo


def _zero_acc_slice(zbuf, acc, sid):
    def zb(i, _):
        zbuf[pl.ds(i * 16, 16)] = jnp.zeros((16,), f32)
        return 0
    lax.fori_loop(0, SL // 16, zb, 0)
    pltpu.sync_copy(zbuf.at[pl.ds(0, SL)], acc.at[pl.ds(sid * SL, SL)])


@functools.partial(
    pl.kernel,
    out_type=jax.ShapeDtypeStruct((NC * NPAD,), f32),
    mesh=_mesh,
    compiler_params=pltpu.CompilerParams(needs_layout_passes=False),
    scratch_types=[
        pltpu.VMEM((2, EPT), i32),
        pltpu.VMEM((1, 128), f32),
        pltpu.VMEM((SL,), f32),
        pltpu.VMEM_SHARED((NPAD,), f32),
        pltpu.SemaphoreType.DMA,
        pltpu.SemaphoreType.DMA,
    ],
)
def _deg_kernel(ei, ones_row, out, ebuf, obuf, zbuf, acc, esem, ssem):
    cid = lax.axis_index("c")
    sid = lax.axis_index("s")
    wid, start, row_lo = _tile_window(cid, sid)
    edma = pltpu.async_copy(ei.at[:, pl.ds(start, EPT)], ebuf, esem)
    pltpu.sync_copy(ones_row, obuf)
    _zero_acc_slice(zbuf, acc, sid)
    edma.wait()
    plsc.subcore_barrier()

    def fire(j, _):
        pltpu.async_copy(obuf.at[0], acc.at[ebuf.at[1, pl.ds(j * 128, 128)]],
                         ssem, add=True)
        return 0
    lax.fori_loop(row_lo, RPT, fire, 0)

    def drain(j, _):
        pltpu.make_async_copy(obuf.at[0],
                              acc.at[ebuf.at[1, pl.ds(j * 128, 128)]],
                              ssem).wait()
        return 0
    lax.fori_loop(row_lo, RPT, drain, 0)
    plsc.subcore_barrier()
    pltpu.sync_copy(acc.at[pl.ds(sid * SL, SL)], zbuf.at[pl.ds(0, SL)])
    pltpu.sync_copy(zbuf.at[pl.ds(0, SL)],
                    out.at[pl.ds(cid * NPAD + sid * SL, SL)])


@functools.partial(
    pl.kernel,
    out_type=jax.ShapeDtypeStruct((NC * NPAD,), f32),
    mesh=_mesh,
    compiler_params=pltpu.CompilerParams(needs_layout_passes=False),
    scratch_types=[
        pltpu.VMEM((NPAD,), f32),
        pltpu.VMEM((2, EPT), i32),
        pltpu.VMEM((EPT,), f32),
        pltpu.VMEM_SHARED((NPAD,), f32),
        pltpu.SemaphoreType.DMA,
        pltpu.SemaphoreType.DMA,
        pltpu.SemaphoreType.DMA,
    ],
)
def _edge_kernel(table, ei, out, tab, ebuf, vbuf, acc, esem, tsem, ssem):
    cid = lax.axis_index("c")
    sid = lax.axis_index("s")
    wid, start, row_lo = _tile_window(cid, sid)
    edma = pltpu.async_copy(ei.at[:, pl.ds(start, EPT)], ebuf, esem)
    tdma = pltpu.async_copy(table, tab, tsem)
    _zero_acc_slice(vbuf, acc, sid)
    edma.wait()
    tdma.wait()

    plsc.subcore_barrier()

    def gf(j, _):
        for u in range(8):
            idx = ebuf[0, pl.ds(j * 128 + u * 16, 16)]
            vbuf[pl.ds(j * 128 + u * 16, 16)] = plsc.load_gather(tab, [idx])
        pltpu.async_copy(vbuf.at[pl.ds(j * 128, 128)],
                         acc.at[ebuf.at[1, pl.ds(j * 128, 128)]],
                         ssem, add=True)
        return 0
    lax.fori_loop(row_lo, RPT, gf, 0)

    def drain(j, _):
        pltpu.make_async_copy(vbuf.at[pl.ds(j * 128, 128)],
                              acc.at[ebuf.at[1, pl.ds(j * 128, 128)]],
                              ssem).wait()
        return 0
    lax.fori_loop(row_lo, RPT, drain, 0)
    plsc.subcore_barrier()
    pltpu.sync_copy(acc.at[pl.ds(sid * SL, SL)], vbuf.at[pl.ds(0, SL)])
    pltpu.sync_copy(vbuf.at[pl.ds(0, SL)],
                    out.at[pl.ds(cid * NPAD + sid * SL, SL)])


def _t1_body(dp, xr, a_o, dis_o, s1self_o, dis2_o):
    d = dp[...]
    deg = d[:R] + d[R:] + 1.0
    dis = lax.rsqrt(deg)
    dis = dis * (1.5 - 0.5 * deg * dis * dis)
    dis2 = 1.0 / deg
    dis_o[...] = dis
    dis2_o[...] = dis2
    a_o[...] = xr[...] * dis
    s1self_o[...] = xr[...] * dis2


_t1 = pl.pallas_call(
    _t1_body,
    out_shape=[jax.ShapeDtypeStruct((R, 128), f32)] * 4,
)

RB = 56


def _t2_body(p0, p1, dis, s1self, dis2, w1s, b1s, w2s, b_o, zself_o):
    s1 = dis[...] * (p0[...] + p1[...]) + s1self[...]

    def step(k, z):
        h = jnp.maximum(s1 * w1s[0, k] + b1s[0, k], 0.0)
        return z + h * w2s[0, k]
    z = lax.fori_loop(0, 128, step, jnp.zeros_like(s1))
    b_o[...] = z * dis[...]
    zself_o[...] = z * dis2[...]


_blk = pl.BlockSpec((RB, 128), lambda i: (i, 0))
_pblk0 = pl.BlockSpec((RB, 128), lambda i: (i, 0))
_pblk1 = pl.BlockSpec((RB, 128), lambda i: (i + R // RB, 0))
_wrow = pl.BlockSpec(memory_space=pltpu.SMEM)
_t2 = pl.pallas_call(
    _t2_body,
    grid=(R // RB,),
    in_specs=[_pblk0, _pblk1, _blk, _blk, _blk, _wrow, _wrow, _wrow],
    out_specs=[_blk, _blk],
    out_shape=[jax.ShapeDtypeStruct((R, 128), f32)] * 2,
)


def _t3_body(op, dis, zself, b2s, out_o):
    o = op[...]
    out_o[...] = dis[...] * (o[:R] + o[R:]) + zself[...] + b2s[0, 0]


_t3 = pl.pallas_call(
    _t3_body,
    in_specs=[pl.BlockSpec(memory_space=pltpu.VMEM)] * 3
    + [pl.BlockSpec(memory_space=pltpu.SMEM)],
    out_shape=jax.ShapeDtypeStruct((R, 128), f32),
)


def kernel(x, edge_index, W1, b1, W2, b2):
    x = x.astype(f32)
    ei = edge_index.astype(i32)
    xp = jnp.pad(x[:, 0], (0, NPAD - N)).reshape(R, 128)
    ones_row = jnp.ones((1, 128), f32)

    degp = _deg_kernel(ei, ones_row).reshape(2 * R, 128)
    a, dis, s1self, dis2 = _t1(degp, xp)

    s1p = _edge_kernel(a.reshape(NPAD), ei).reshape(2 * R, 128)
    bvec, zself = _t2(s1p, s1p, dis, s1self, dis2,
                      W1, b1.reshape(1, 128), W2.reshape(1, 128))

    outp = _edge_kernel(bvec.reshape(NPAD), ei).reshape(2 * R, 128)
    out = _t3(outp, dis, zself, b2.reshape(1, 1))
    return out.reshape(NPAD)[:N][:, None]


# speedup vs baseline: 265.8880x; 1.0111x over previous
import functools

import jax
import jax.numpy as jnp
from jax import lax
from jax.experimental import pallas as pl
from jax.experimental.pallas import tpu as pltpu
from jax.experimental.pallas import tpu_sc as plsc

f32 = jnp.float32
i32 = jnp.int32

N = 50000
NPAD = 50176
R = NPAD // 128
E = 800000
NC, NS = 2, 16
NT = NC * NS
EPT = 25088
RPT = EPT // 128
LAST_START = E - EPT
SKIP_LAST = 22
SL = NPAD // NS

_mesh = plsc.VectorSubcoreMesh(
    core_axis_name="c", subcore_axis_name="s", num_cores=NC, num_subcores=NS)


def _tile_window(cid, sid):
    wid = cid * NS + sid
    start = jnp.minimum(wid * EPT, LAST_START)
    row_lo = jnp.where(wid == NT - 1, SKIP_LAST, 0)
    return wid, start, row_lo


def _zero_acc_slice(zbuf, acc, sid):
    def zb(i, _):
        zbuf[pl.ds(i * 16, 16)] = jnp.zeros((16,), f32)
        return 0
    lax.fori_loop(0, SL // 16, zb, 0)
    pltpu.sync_copy(zbuf.at[pl.ds(0, SL)], acc.at[pl.ds(sid * SL, SL)])


@functools.partial(
    pl.kernel,
    out_type=jax.ShapeDtypeStruct((NC * NPAD,), f32),
    mesh=_mesh,
    compiler_params=pltpu.CompilerParams(needs_layout_passes=False),
    scratch_types=[
        pltpu.VMEM((2, EPT), i32),
        pltpu.VMEM((1, 128), f32),
        pltpu.VMEM((SL,), f32),
        pltpu.VMEM_SHARED((NPAD,), f32),
        pltpu.SemaphoreType.DMA,
        pltpu.SemaphoreType.DMA,
    ],
)
def _deg_kernel(ei, ones_row, out, ebuf, obuf, zbuf, acc, esem, ssem):
    cid = lax.axis_index("c")
    sid = lax.axis_index("s")
    wid, start, row_lo = _tile_window(cid, sid)
    edma = pltpu.async_copy(ei.at[:, pl.ds(start, EPT)], ebuf, esem)
    pltpu.sync_copy(ones_row, obuf)
    _zero_acc_slice(zbuf, acc, sid)
    edma.wait()
    plsc.subcore_barrier()

    def fire(j, _):
        pltpu.async_copy(obuf.at[0], acc.at[ebuf.at[1, pl.ds(j * 128, 128)]],
                         ssem, add=True)
        return 0
    lax.fori_loop(row_lo, RPT, fire, 0)

    def drain(j, _):
        pltpu.make_async_copy(obuf.at[0],
                              acc.at[ebuf.at[1, pl.ds(j * 128, 128)]],
                              ssem).wait()
        return 0
    lax.fori_loop(row_lo, RPT, drain, 0)
    plsc.subcore_barrier()
    pltpu.sync_copy(acc.at[pl.ds(sid * SL, SL)], zbuf.at[pl.ds(0, SL)])
    pltpu.sync_copy(zbuf.at[pl.ds(0, SL)],
                    out.at[pl.ds(cid * NPAD + sid * SL, SL)])


def _rsqrt16(d):
    y = plsc.bitcast(0x5F3759DF - (plsc.bitcast(d, i32) >> 1), f32)
    for _ in range(3):
        y = y * (1.5 - 0.5 * d * y * y)
    return y


@functools.partial(
    pl.kernel,
    out_type=[jax.ShapeDtypeStruct((NC * NPAD,), f32),
              jax.ShapeDtypeStruct((NC * NPAD,), f32)],
    mesh=_mesh,
    compiler_params=pltpu.CompilerParams(needs_layout_passes=False),
    scratch_types=[
        pltpu.VMEM((NPAD,), f32),
        pltpu.VMEM((2, EPT), i32),
        pltpu.VMEM((EPT,), f32),
        pltpu.VMEM_SHARED((NPAD,), f32),
        pltpu.SemaphoreType.DMA,
        pltpu.SemaphoreType.DMA,
        pltpu.SemaphoreType.DMA,
    ],
)
def _s1_kernel(degp, xp, ei, out, a_hbm, tab, ebuf, vbuf, acc,
               esem, tsem, ssem):
    cid = lax.axis_index("c")
    sid = lax.axis_index("s")
    wid, start, row_lo = _tile_window(cid, sid)
    edma = pltpu.async_copy(ei.at[:, pl.ds(start, EPT)], ebuf, esem)
    d0ma = pltpu.async_copy(degp.at[pl.ds(sid * SL, SL)],
                            vbuf.at[pl.ds(0, SL)], tsem)
    d1ma = pltpu.async_copy(degp.at[pl.ds(NPAD + sid * SL, SL)],
                            vbuf.at[pl.ds(SL, SL)], tsem)
    xma = pltpu.async_copy(xp.at[pl.ds(sid * SL, SL)],
                           vbuf.at[pl.ds(2 * SL, SL)], tsem)
    def zb(i, _):
        vbuf[pl.ds(3 * SL + i * 16, 16)] = jnp.zeros((16,), f32)
        return 0
    lax.fori_loop(0, SL // 16, zb, 0)
    pltpu.sync_copy(vbuf.at[pl.ds(3 * SL, SL)], acc.at[pl.ds(sid * SL, SL)])
    d0ma.wait()
    d1ma.wait()
    xma.wait()

    def ab(i, _):
        d = vbuf[pl.ds(i * 16, 16)] + vbuf[pl.ds(SL + i * 16, 16)] + 1.0
        vbuf[pl.ds(3 * SL + i * 16, 16)] = (
            vbuf[pl.ds(2 * SL + i * 16, 16)] * _rsqrt16(d))
        return 0
    lax.fori_loop(0, SL // 16, ab, 0)
    pltpu.sync_copy(vbuf.at[pl.ds(3 * SL, SL)],
                    a_hbm.at[pl.ds(cid * NPAD + sid * SL, SL)])
    plsc.subcore_barrier()
    pltpu.sync_copy(a_hbm.at[pl.ds(cid * NPAD, NPAD)], tab)
    edma.wait()

    def gf(j, _):
        for u in range(8):
            idx = ebuf[0, pl.ds(j * 128 + u * 16, 16)]
            vbuf[pl.ds(j * 128 + u * 16, 16)] = plsc.load_gather(tab, [idx])
        pltpu.async_copy(vbuf.at[pl.ds(j * 128, 128)],
                         acc.at[ebuf.at[1, pl.ds(j * 128, 128)]],
                         ssem, add=True)
        return 0
    lax.fori_loop(row_lo, RPT, gf, 0)

    def drain(j, _):
        pltpu.make_async_copy(vbuf.at[pl.ds(j * 128, 128)],
                              acc.at[ebuf.at[1, pl.ds(j * 128, 128)]],
                              ssem).wait()
        return 0
    lax.fori_loop(row_lo, RPT, drain, 0)
    plsc.subcore_barrier()
    pltpu.sync_copy(acc.at[pl.ds(sid * SL, SL)], vbuf.at[pl.ds(0, SL)])
    pltpu.sync_copy(vbuf.at[pl.ds(0, SL)],
                    out.at[pl.ds(cid * NPAD + sid * SL, SL)])


@functools.partial(
    pl.kernel,
    out_type=jax.ShapeDtypeStruct((NC * NPAD,), f32),
    mesh=_mesh,
    compiler_params=pltpu.CompilerParams(needs_layout_passes=False),
    scratch_types=[
        pltpu.VMEM((NPAD,), f32),
        pltpu.VMEM((2, EPT), i32),
        pltpu.VMEM((EPT,), f32),
        pltpu.VMEM_SHARED((NPAD,), f32),
        pltpu.SemaphoreType.DMA,
        pltpu.SemaphoreType.DMA,
        pltpu.SemaphoreType.DMA,
    ],
)
def _out_kernel(table, ei, out, tab, ebuf, vbuf, acc, esem, tsem, ssem):
    cid = lax.axis_index("c")
    sid = lax.axis_index("s")
    wid, start, row_lo = _tile_window(cid, sid)
    edma = pltpu.async_copy(ei.at[:, pl.ds(start, EPT)], ebuf, esem)
    tdma = pltpu.async_copy(table, tab, tsem)
    _zero_acc_slice(vbuf, acc, sid)
    edma.wait()
    tdma.wait()
    plsc.subcore_barrier()

    def gf(j, _):
        for u in range(8):
            idx = ebuf[0, pl.ds(j * 128 + u * 16, 16)]
            vbuf[pl.ds(j * 128 + u * 16, 16)] = plsc.load_gather(tab, [idx])
        pltpu.async_copy(vbuf.at[pl.ds(j * 128, 128)],
                         acc.at[ebuf.at[1, pl.ds(j * 128, 128)]],
                         ssem, add=True)
        return 0
    lax.fori_loop(row_lo, RPT, gf, 0)

    def drain(j, _):
        pltpu.make_async_copy(vbuf.at[pl.ds(j * 128, 128)],
                              acc.at[ebuf.at[1, pl.ds(j * 128, 128)]],
                              ssem).wait()
        return 0
    lax.fori_loop(row_lo, RPT, drain, 0)
    plsc.subcore_barrier()
    pltpu.sync_copy(acc.at[pl.ds(sid * SL, SL)], vbuf.at[pl.ds(0, SL)])
    pltpu.sync_copy(vbuf.at[pl.ds(0, SL)],
                    out.at[pl.ds(cid * NPAD + sid * SL, SL)])


def _tc_dis(q0, q1):
    deg = q0 + q1 + 1.0
    dis = lax.rsqrt(deg)
    dis = dis * (1.5 - 0.5 * deg * dis * dis)
    return deg, dis


RB = 56


def _t2_body(p0, p1, q0, q1, xb, w1s, b1s, w2s, b_o, zself_o):
    deg, dis = _tc_dis(q0[...], q1[...])
    dis2 = 1.0 / deg
    s1 = dis * (p0[...] + p1[...]) + xb[...] * dis2

    def step(k, z):
        h = jnp.maximum(s1 * w1s[0, k] + b1s[0, k], 0.0)
        return z + h * w2s[0, k]
    z = lax.fori_loop(0, 128, step, jnp.zeros_like(s1))
    b_o[...] = z * dis
    zself_o[...] = z * dis2


_blk = pl.BlockSpec((RB, 128), lambda i: (i, 0))
_pblk0 = pl.BlockSpec((RB, 128), lambda i: (i, 0))
_pblk1 = pl.BlockSpec((RB, 128), lambda i: (i + R // RB, 0))
_wrow = pl.BlockSpec(memory_space=pltpu.SMEM)
_t2 = pl.pallas_call(
    _t2_body,
    grid=(R // RB,),
    in_specs=[_pblk0, _pblk1, _pblk0, _pblk1, _blk, _wrow, _wrow, _wrow],
    out_specs=[_blk, _blk],
    out_shape=[jax.ShapeDtypeStruct((R, 128), f32)] * 2,
)


def _t3_body(op, qp, zself, b2s, out_o):
    o = op[...]
    q = qp[...]
    _, dis = _tc_dis(q[:R], q[R:])
    out_o[...] = dis * (o[:R] + o[R:]) + zself[...] + b2s[0, 0]


_t3 = pl.pallas_call(
    _t3_body,
    in_specs=[pl.BlockSpec(memory_space=pltpu.VMEM)] * 3
    + [pl.BlockSpec(memory_space=pltpu.SMEM)],
    out_shape=jax.ShapeDtypeStruct((R, 128), f32),
)


def kernel(x, edge_index, W1, b1, W2, b2):
    x = x.astype(f32)
    ei = edge_index.astype(i32)
    xp = jnp.pad(x[:, 0], (0, NPAD - N))
    ones_row = jnp.ones((1, 128), f32)

    degp = _deg_kernel(ei, ones_row)
    degp2 = degp.reshape(2 * R, 128)

    s1p, _ = _s1_kernel(degp, xp, ei)
    bvec, zself = _t2(s1p.reshape(2 * R, 128), s1p.reshape(2 * R, 128),
                      degp2, degp2, xp.reshape(R, 128),
                      W1, b1.reshape(1, 128), W2.reshape(1, 128))

    outp = _out_kernel(bvec.reshape(NPAD), ei).reshape(2 * R, 128)
    out = _t3(outp, degp2, zself, b2.reshape(1, 1))
    return out.reshape(NPAD)[:N][:, None]
